# Initial kernel scaffold; baseline (speedup 1.0000x reference)
import jax
import jax.numpy as jnp
from jax.experimental import pallas as pl


def kernel(coors_kspace, Wx, Wy):
    raise NotImplementedError("write your pallas kernel here")


# speedup vs baseline: 4.2769x; 4.2769x over previous
import functools

import jax
import jax.numpy as jnp
from jax import lax
from jax.experimental import pallas as pl
from jax.experimental.pallas import tpu as pltpu
from jax.experimental.pallas import tpu_sc as plsc

N = 1048576
NUM_X = 320
NUM_Y = 320
EMB = 3
IN_D = 4
OUT_D = 2 * EMB + 2

_info = plsc.get_sparse_core_info()
NC = _info.num_cores
NS = _info.num_subcores
L = _info.num_lanes
NW = NC * NS

ROWS_PER_W = N // NW
CHUNK = 2048
NCHUNK = ROWS_PER_W // CHUNK


def _body(coors_hbm, wx_hbm, wy_hbm, out_hbm,
          wx_v, wy_v, coors_v, out_v, sem_in, sem_out):
    wid = lax.axis_index("s") * NC + lax.axis_index("c")
    base_w = wid * ROWS_PER_W

    pltpu.sync_copy(wx_hbm, wx_v)
    pltpu.sync_copy(wy_hbm, wy_v)

    iota = lax.iota(jnp.int32, L)
    r4_0 = iota * IN_D
    r8_0 = iota * OUT_D

    def chunk_body(ci, carry):
        base = base_w + ci * CHUNK
        pltpu.sync_copy(coors_hbm.at[pl.ds(base * IN_D, CHUNK * IN_D)],
                        coors_v)

        def grp(g, rr):
            r4, r8 = rr
            c0 = plsc.load_gather(coors_v, [r4])
            c1 = plsc.load_gather(coors_v, [r4 + 1])
            c2 = plsc.load_gather(coors_v, [r4 + 2])
            c3 = plsc.load_gather(coors_v, [r4 + 3])
            ix3 = c0.astype(jnp.int32) * EMB
            iy3 = c1.astype(jnp.int32) * EMB
            wx0 = plsc.load_gather(wx_v, [ix3])
            wx1 = plsc.load_gather(wx_v, [ix3 + 1])
            wx2 = plsc.load_gather(wx_v, [ix3 + 2])
            wy0 = plsc.load_gather(wy_v, [iy3])
            wy1 = plsc.load_gather(wy_v, [iy3 + 1])
            wy2 = plsc.load_gather(wy_v, [iy3 + 2])
            plsc.store_scatter(out_v, [r8], wx0)
            plsc.store_scatter(out_v, [r8 + 1], wx1)
            plsc.store_scatter(out_v, [r8 + 2], wx2)
            plsc.store_scatter(out_v, [r8 + 3], wy0)
            plsc.store_scatter(out_v, [r8 + 4], wy1)
            plsc.store_scatter(out_v, [r8 + 5], wy2)
            plsc.store_scatter(out_v, [r8 + 6], c2)
            plsc.store_scatter(out_v, [r8 + 7], c3)
            return (r4 + L * IN_D, r8 + L * OUT_D)

        lax.fori_loop(0, CHUNK // L, grp, (r4_0, r8_0))
        pltpu.sync_copy(out_v, out_hbm.at[pl.ds(base * OUT_D, CHUNK * OUT_D)])
        return carry

    lax.fori_loop(0, NCHUNK, chunk_body, 0)


@jax.jit
def _features(coors_flat, wx_flat, wy_flat):
    mesh = plsc.VectorSubcoreMesh(core_axis_name="c", subcore_axis_name="s")
    run = functools.partial(
        pl.kernel,
        out_type=jax.ShapeDtypeStruct((N * OUT_D,), jnp.float32),
        mesh=mesh,
        compiler_params=pltpu.CompilerParams(needs_layout_passes=False),
        scratch_types=[
            pltpu.VMEM((NUM_X * EMB,), jnp.float32),
            pltpu.VMEM((NUM_Y * EMB,), jnp.float32),
            pltpu.VMEM((CHUNK * IN_D,), jnp.float32),
            pltpu.VMEM((CHUNK * OUT_D,), jnp.float32),
            pltpu.SemaphoreType.DMA,
            pltpu.SemaphoreType.DMA,
        ],
    )(_body)
    return run(coors_flat, wx_flat, wy_flat)


def kernel(coors_kspace, Wx, Wy):
    flat = _features(coors_kspace.reshape(-1), Wx.reshape(-1), Wy.reshape(-1))
    return (flat.reshape(N, OUT_D), Wx, Wy)
